# baseline (speedup 1.0000x reference)
import functools

import jax
import jax.numpy as jnp
from jax import lax
from jax.experimental import pallas as pl
from jax.experimental.pallas import tpu as pltpu
from jax.experimental.pallas import tpu_sc as plsc

NC = 2
NS = 16
L = 16
NW = NC * NS


def _cofm_body(b_per_w, d_model,
               u_ids_hbm, i_ids_hbm, user_emb_hbm, item_emb_hbm,
               user_bias_hbm, item_bias_hbm, bias_hbm, out_hbm,
               uid_v, iid_v, urows_v, irows_v, ub_v, ib_v, bias_v, out_v,
               sem_rows, sem_bias):
  wid = lax.axis_index("s") * NC + lax.axis_index("c")
  base = wid * b_per_w

  pltpu.sync_copy(u_ids_hbm.at[pl.ds(base, b_per_w)], uid_v)
  pltpu.sync_copy(i_ids_hbm.at[pl.ds(base, b_per_w)], iid_v)

  cp_u = pltpu.async_copy(user_emb_hbm.at[uid_v], urows_v, sem_rows)
  cp_i = pltpu.async_copy(item_emb_hbm.at[iid_v], irows_v, sem_rows)
  cp_ub = pltpu.async_copy(user_bias_hbm.at[uid_v], ub_v, sem_bias)
  cp_ib = pltpu.async_copy(item_bias_hbm.at[iid_v], ib_v, sem_bias)
  pltpu.sync_copy(bias_hbm, bias_v)
  cp_u.wait()
  cp_i.wait()
  cp_ub.wait()
  cp_ib.wait()

  lanes = lax.iota(jnp.int32, L)
  bias_splat = bias_v[...]
  cols = [lanes * 0 + d for d in range(d_model)]

  def group(g, carry):
    row = g * L
    rows = lanes + row
    acc = ub_v[pl.ds(row, L)] + ib_v[pl.ds(row, L)] + bias_splat
    for d in range(d_model):
      acc = acc + (plsc.load_gather(urows_v, [rows, cols[d]]) *
                   plsc.load_gather(irows_v, [rows, cols[d]]))
    out_v[pl.ds(row, L)] = acc
    return carry

  lax.fori_loop(0, b_per_w // L, group, 0)

  pltpu.sync_copy(out_v, out_hbm.at[pl.ds(base, b_per_w)])


def kernel(u_ids, i_ids, user_emb, item_emb, user_bias, item_bias, bias):
  batch = u_ids.shape[0]
  d_model = user_emb.shape[1]
  b_per_w = batch // NW
  bias16 = jnp.broadcast_to(bias, (L,))

  mesh = plsc.VectorSubcoreMesh(core_axis_name="c", subcore_axis_name="s",
                                num_cores=NC, num_subcores=NS)
  run = pl.kernel(
      functools.partial(_cofm_body, b_per_w, d_model),
      out_type=jax.ShapeDtypeStruct((batch,), jnp.float32),
      mesh=mesh,
      compiler_params=pltpu.CompilerParams(needs_layout_passes=False,
                                           use_tc_tiling_on_sc=False),
      scratch_types=[
          pltpu.VMEM((b_per_w,), jnp.int32),
          pltpu.VMEM((b_per_w,), jnp.int32),
          pltpu.VMEM((b_per_w, d_model), jnp.float32),
          pltpu.VMEM((b_per_w, d_model), jnp.float32),
          pltpu.VMEM((b_per_w,), jnp.float32),
          pltpu.VMEM((b_per_w,), jnp.float32),
          pltpu.VMEM((L,), jnp.float32),
          pltpu.VMEM((b_per_w,), jnp.float32),
          pltpu.SemaphoreType.DMA,
          pltpu.SemaphoreType.DMA,
      ],
  )
  return run(u_ids, i_ids, user_emb, item_emb, user_bias, item_bias, bias16)
